# baseline (speedup 1.0000x reference)
import functools

import jax
import jax.numpy as jnp
from jax import lax
from jax.experimental import pallas as pl
from jax.experimental.pallas import tpu as pltpu
from jax.experimental.pallas import tpu_sc as plsc

_TOKENS = 8192
_DIM = 2048
_NE = 16
_EPS = 1e-06
_CAP = float(_TOKENS)
_MM_TILE = 1024
_MM_GRID = _TOKENS // _MM_TILE
_NWORKERS = 32
_TPW = _TOKENS // _NWORKERS


def _mm_body(x_ref, w_ref, b_ref, out_ref):
    out_ref[...] = lax.dot_general(
        x_ref[...], w_ref[...], (((1,), (1,)), ((), ())),
        preferred_element_type=jnp.float32,
    ) + b_ref[...]


def _route_body(logits_hbm, out_hbm, lv, mv):
    wid = lax.axis_index("s") * 2 + lax.axis_index("c")
    base = wid * _TPW
    pltpu.sync_copy(logits_hbm.at[pl.ds(base, _TPW)], lv)

    def body(i, carry):
        v = lv[i, :]
        m = jnp.max(v)
        e = jnp.exp(v - m)
        p = e / jnp.sum(e)
        pm = jnp.max(p)
        idx = lax.iota(jnp.int32, 16)
        first = jnp.min(jnp.where(p == pm, idx, _NE))
        mv[i, :] = jnp.where(idx == first, p, 0.0)
        return carry

    lax.fori_loop(0, _TPW, body, 0)
    pltpu.sync_copy(mv, out_hbm.at[pl.ds(base, _TPW)])


def _fin_body(m_ref, out_ref):
    m = m_ref[...]
    denom = jnp.sum(m, axis=0, keepdims=True) + _EPS
    out_ref[...] = m * (_CAP / denom)


def kernel(x, W, b):
    b2 = b.reshape(1, _NE)
    logits = pl.pallas_call(
        _mm_body,
        grid=(_MM_GRID,),
        in_specs=[
            pl.BlockSpec((_MM_TILE, _DIM), lambda i: (i, 0)),
            pl.BlockSpec((_NE, _DIM), lambda i: (0, 0)),
            pl.BlockSpec((1, _NE), lambda i: (0, 0)),
        ],
        out_specs=pl.BlockSpec((_MM_TILE, _NE), lambda i: (i, 0)),
        out_shape=jax.ShapeDtypeStruct((_TOKENS, _NE), jnp.float32),
    )(x, W, b2)

    route = pl.kernel(
        _route_body,
        mesh=plsc.VectorSubcoreMesh(core_axis_name="c", subcore_axis_name="s"),
        compiler_params=pltpu.CompilerParams(needs_layout_passes=False),
        out_type=jax.ShapeDtypeStruct((_TOKENS, _NE), jnp.float32),
        scratch_types=[
            pltpu.VMEM((_TPW, _NE), jnp.float32),
            pltpu.VMEM((_TPW, _NE), jnp.float32),
        ],
    )
    masked = route(logits)

    return pl.pallas_call(
        _fin_body,
        grid=(1,),
        in_specs=[pl.BlockSpec((_TOKENS, _NE), lambda i: (0, 0))],
        out_specs=pl.BlockSpec((_TOKENS, _NE), lambda i: (0, 0)),
        out_shape=jax.ShapeDtypeStruct((_TOKENS, _NE), jnp.float32),
    )(masked)
